# baseline (speedup 1.0000x reference)
import functools

import jax
import jax.numpy as jnp
from jax import lax
from jax.experimental import pallas as pl
from jax.experimental.pallas import tpu as pltpu
from jax.experimental.pallas import tpu_sc as plsc

_B, _N, _D, _L = 16, 2048, 512, 512
_NS = _N // 2
_K = min(int(2048 * 0.3), _NS)
_NNEW = _N - _K
_TOT = _B * _NNEW
_CH = 32
_NCHUNK = _TOT // _CH
_NW = 32
_CPW = -(-_NCHUNK // _NW)

_HI = jax.lax.Precision.HIGHEST


def _dot(x, y):
    return jax.lax.dot_general(
        x, y, (((1,), (0,)), ((), ())),
        precision=_HI, preferred_element_type=jnp.float32)


def _metric_body(xfr_ref, me_ref, mo_ref):
    blk = xfr_ref[0]
    me_ref[0] = jnp.sum(blk[:, :_D], axis=1, keepdims=True)
    mo_ref[0] = jnp.sum(blk[:, _D:], axis=1, keepdims=True)


def _metric(xfr):
    return pl.pallas_call(
        _metric_body,
        grid=(_B,),
        in_specs=[pl.BlockSpec((1, _NS, 2 * _D), lambda b: (b, 0, 0))],
        out_specs=[pl.BlockSpec((1, _NS, 1), lambda b: (b, 0, 0)),
                   pl.BlockSpec((1, _NS, 1), lambda b: (b, 0, 0))],
        out_shape=[jax.ShapeDtypeStruct((_B, _NS, 1), jnp.float32),
                   jax.ShapeDtypeStruct((_B, _NS, 1), jnp.float32)],
    )(xfr)


def _plan_body(me_ref, mo_ref, ue_ref, uo_ref, pg_ref, mk_ref, cn_ref,
               sl_ref, dp_ref):
    b = pl.program_id(0)
    ns = _NS
    f32 = jnp.float32

    mean_e = me_ref[0] / f32(_D)
    mean_o = mo_ref[0] / f32(_D)
    norm2 = jnp.sum(mean_e * mean_e, keepdims=True) + \
        jnp.sum(mean_o * mean_o, keepdims=True)
    den = jnp.maximum(jnp.sqrt(norm2), f32(1e-12))
    a_col = mean_e / den
    b_col = mean_o / den

    iota_col = lax.broadcasted_iota(jnp.int32, (ns, 1), 0).astype(f32)
    iota_row = lax.broadcasted_iota(jnp.int32, (1, ns), 1).astype(f32)
    R = lax.broadcasted_iota(jnp.int32, (ns, ns), 0).astype(f32)
    C = lax.broadcasted_iota(jnp.int32, (ns, ns), 1).astype(f32)
    eye = (R == C).astype(f32)
    ones_row = jnp.ones((1, ns), f32)
    ones_col = jnp.ones((ns, 1), f32)

    def to_row(v_col):
        return _dot(ones_row, eye * v_col)

    bmax = jnp.max(b_col, keepdims=True)
    bmin = jnp.min(b_col, keepdims=True)
    big = f32(ns + 1)
    jmax = jnp.min(jnp.where(b_col == bmax, iota_col, big), keepdims=True)
    jmin = jnp.min(jnp.where(b_col == bmin, iota_col, big), keepdims=True)

    zero = jnp.zeros((1, 1), f32)
    pos = a_col > 0
    neg = a_col < 0
    v_col = jnp.where(pos, a_col * bmax,
                      jnp.where(neg, a_col * bmin, zero))
    nidx_col = jnp.where(pos, jmax, jnp.where(neg, jmin, zero))

    v_row = to_row(v_col)
    gt = (v_row > v_col).astype(f32)
    eqlt = ((v_row == v_col) & (C < R)).astype(f32)
    rank_col = _dot(gt + eqlt, ones_col)
    sel = rank_col < f32(_K)
    sel_col = sel.astype(f32)

    tril = (C <= R).astype(f32)
    cum_incl = _dot(tril, sel_col)
    cum_excl = cum_incl - sel_col

    def cum_at(t):
        return jnp.sum(jnp.where(iota_col == t, cum_incl, zero),
                       keepdims=True)

    cmax = cum_at(jmax)
    cmin = cum_at(jmin)
    c0 = cum_at(zero)
    cum_at_nidx = jnp.where(pos, cmax, jnp.where(neg, cmin, c0))

    i2_col = 2.0 * iota_col
    dst_col = 2.0 * nidx_col + 1.0
    ue = jnp.where(sel, dst_col - cum_at_nidx, i2_col - cum_excl)
    uo = i2_col + 1.0 - cum_incl
    ue_ref[0] = ue.astype(jnp.int32)
    uo_ref[0] = uo.astype(jnp.int32)

    rank_even = i2_col - cum_excl
    rank_odd = i2_col + 1.0 - cum_incl
    keep_col = 1.0 - sel_col
    C2 = lax.broadcasted_iota(jnp.int32, (ns, _NNEW), 1).astype(f32)
    e1t = (rank_even == C2).astype(f32) * keep_col
    e2t = (rank_odd == C2).astype(f32)
    i2_row = 2.0 * iota_row
    p_row = _dot(i2_row, e1t) + _dot(i2_row + 1.0, e2t)
    pg_ref[0] = (p_row + f32(_N) * b.astype(f32)).astype(jnp.int32)

    sel_row = to_row(sel_col)
    nidx_row = to_row(nidx_col)
    t_list = [jmax, jmin, zero, zero]
    c_list = [cmax, cmin, c0, c0]
    masks = []
    cnts = []
    slots = []
    for t, ct in zip(t_list, c_list):
        m = sel_row * (nidx_row == t).astype(f32)
        masks.append(m)
        cnts.append(1.0 + jnp.sum(m, keepdims=True))
        slots.append(2.0 * t + 1.0 - ct + f32(_NNEW) * b.astype(f32))
    mk_ref[0] = jnp.concatenate(masks, axis=0)
    cn_ref[0] = jnp.concatenate([c.reshape(1, 1) for c in cnts], axis=0)
    sl_ref[0] = jnp.concatenate(slots, axis=1).astype(jnp.int32)
    dp_ref[0] = jnp.concatenate(t_list, axis=1).astype(jnp.int32)


def _plan(me, mo):
    return pl.pallas_call(
        _plan_body,
        grid=(_B,),
        in_specs=[pl.BlockSpec((1, _NS, 1), lambda b: (b, 0, 0)),
                  pl.BlockSpec((1, _NS, 1), lambda b: (b, 0, 0))],
        out_specs=[pl.BlockSpec((1, _NS, 1), lambda b: (b, 0, 0)),
                   pl.BlockSpec((1, _NS, 1), lambda b: (b, 0, 0)),
                   pl.BlockSpec((1, 1, _NNEW), lambda b: (b, 0, 0)),
                   pl.BlockSpec((1, 4, _NS), lambda b: (b, 0, 0)),
                   pl.BlockSpec((1, 4, 1), lambda b: (b, 0, 0)),
                   pl.BlockSpec((1, 1, 4), lambda b: (b, 0, 0)),
                   pl.BlockSpec((1, 1, 4), lambda b: (b, 0, 0))],
        out_shape=[jax.ShapeDtypeStruct((_B, _NS, 1), jnp.int32),
                   jax.ShapeDtypeStruct((_B, _NS, 1), jnp.int32),
                   jax.ShapeDtypeStruct((_B, 1, _NNEW), jnp.int32),
                   jax.ShapeDtypeStruct((_B, 4, _NS), jnp.float32),
                   jax.ShapeDtypeStruct((_B, 4, 1), jnp.float32),
                   jax.ShapeDtypeStruct((_B, 1, 4), jnp.int32),
                   jax.ShapeDtypeStruct((_B, 1, 4), jnp.int32)],
    )(me, mo)


def _special_body(dp_ref, xe_ref, re_ref, f0, f1, f2, f3, r0, r1, r2, r3,
                  mk_ref, cn_ref, sf_ref, sr_ref):
    mk = mk_ref[0]
    cnt = cn_ref[0] + jnp.float32(1e-6)
    sfe = _dot(mk, xe_ref[0])
    sra = _dot(mk, re_ref[0])
    rows_f = jnp.concatenate(
        [f0[0, 0, 0], f1[0, 0, 0], f2[0, 0, 0], f3[0, 0, 0]], axis=0)
    rows_r = jnp.concatenate(
        [r0[0, 0, 0], r1[0, 0, 0], r2[0, 0, 0], r3[0, 0, 0]], axis=0)
    sf_ref[0] = (rows_f + sfe) / cnt
    sr_ref[0] = (rows_r + sra) / cnt


def _special(dpair, xfr, xrr, masks, cnts):
    xf5 = xfr.reshape(_B, _NS, 2, 1, _D)
    xr5 = xrr.reshape(_B, _NS, 2, 1, _L)

    def row_spec(j):
        return pl.BlockSpec(
            (1, 1, 1, 1, _D), lambda b, dp, j=j: (b, dp[b, 0, j], 1, 0, 0))

    grid_spec = pltpu.PrefetchScalarGridSpec(
        num_scalar_prefetch=1,
        grid=(_B,),
        in_specs=[pl.BlockSpec((1, _NS, _D), lambda b, dp: (b, 0, 0)),
                  pl.BlockSpec((1, _NS, _L), lambda b, dp: (b, 0, 0)),
                  row_spec(0), row_spec(1), row_spec(2), row_spec(3),
                  row_spec(0), row_spec(1), row_spec(2), row_spec(3),
                  pl.BlockSpec((1, 4, _NS), lambda b, dp: (b, 0, 0)),
                  pl.BlockSpec((1, 4, 1), lambda b, dp: (b, 0, 0))],
        out_specs=[pl.BlockSpec((1, 4, _D), lambda b, dp: (b, 0, 0)),
                   pl.BlockSpec((1, 4, _L), lambda b, dp: (b, 0, 0))],
    )
    return pl.pallas_call(
        _special_body,
        grid_spec=grid_spec,
        out_shape=[jax.ShapeDtypeStruct((_B, 4, _D), jnp.float32),
                   jax.ShapeDtypeStruct((_B, 4, _L), jnp.float32)],
    )(dpair, xfr, xrr, xf5, xf5, xf5, xf5, xr5, xr5, xr5, xr5, masks, cnts)


def _sc_gather(xf2d, xr2d, idx):
    mesh = plsc.VectorSubcoreMesh(core_axis_name="c", subcore_axis_name="s")

    @functools.partial(
        pl.kernel,
        mesh=mesh,
        out_type=(jax.ShapeDtypeStruct((_TOT, _D), jnp.float32),
                  jax.ShapeDtypeStruct((_TOT, _L), jnp.float32)),
        scratch_types=[pltpu.VMEM((_CH,), jnp.int32),
                       pltpu.VMEM((_CH, _D), jnp.float32),
                       pltpu.VMEM((_CH, _L), jnp.float32),
                       pltpu.SemaphoreType.DMA,
                       pltpu.SemaphoreType.DMA],
    )
    def gk(xf_hbm, xr_hbm, idx_hbm, of_hbm, or_hbm, idx_v, rf_v, rr_v,
           smf, smr):
        wid = lax.axis_index("s") * 2 + lax.axis_index("c")

        @pl.loop(0, _CPW)
        def _(j):
            c = wid * _CPW + j

            @pl.when(c < _NCHUNK)
            def _():
                base = c * _CH
                pltpu.sync_copy(idx_hbm.at[pl.ds(base, _CH)], idx_v)
                h1 = pltpu.async_copy(xf_hbm.at[idx_v], rf_v, smf)
                h2 = pltpu.async_copy(xr_hbm.at[idx_v], rr_v, smr)
                h1.wait()
                h2.wait()
                pltpu.sync_copy(rf_v, of_hbm.at[pl.ds(base, _CH)])
                pltpu.sync_copy(rr_v, or_hbm.at[pl.ds(base, _CH)])

    return gk(xf2d, xr2d, idx)


def _overwrite_body(sl_ref, gf_ref, gr_ref, sf_ref, sr_ref, of_ref, or_ref):
    del gf_ref, gr_ref
    of_ref[...] = sf_ref[...]
    or_ref[...] = sr_ref[...]


def _overwrite(slots_flat, gf3, gr3, sf3, sr3):
    dyn = pl.BlockSpec((1, 1, _D), lambda g, sl: (sl[g], 0, 0))
    grid_spec = pltpu.PrefetchScalarGridSpec(
        num_scalar_prefetch=1,
        grid=(_B * 4,),
        in_specs=[dyn, dyn,
                  pl.BlockSpec((1, 1, _D), lambda g, sl: (g, 0, 0)),
                  pl.BlockSpec((1, 1, _L), lambda g, sl: (g, 0, 0))],
        out_specs=[dyn, dyn],
    )
    return pl.pallas_call(
        _overwrite_body,
        grid_spec=grid_spec,
        out_shape=[jax.ShapeDtypeStruct((_TOT, 1, _D), jnp.float32),
                   jax.ShapeDtypeStruct((_TOT, 1, _L), jnp.float32)],
        input_output_aliases={1: 0, 2: 1},
    )(slots_flat, gf3, gr3, sf3, sr3)


@jax.jit
def kernel(x_feat, x_raw):
    xfr = x_feat.reshape(_B, _NS, 2 * _D)
    xrr = x_raw.reshape(_B, _NS, 2 * _L)

    me, mo = _metric(xfr)
    ue, uo, pglob, masks, cnts, slots, dpair = _plan(me, mo)
    unpool = jnp.concatenate([ue, uo], axis=2).reshape(_B, _N)

    sf, sr = _special(dpair, xfr, xrr, masks, cnts)

    gf, gr = _sc_gather(x_feat.reshape(_B * _N, _D),
                        x_raw.reshape(_B * _N, _L),
                        pglob.reshape(_TOT))

    of, orr = _overwrite(slots.reshape(_B * 4),
                         gf.reshape(_TOT, 1, _D),
                         gr.reshape(_TOT, 1, _L),
                         sf.reshape(_B * 4, 1, _D),
                         sr.reshape(_B * 4, 1, _L))

    return (of.reshape(_B, _NNEW, _D),
            orr.reshape(_B, _NNEW, _L),
            unpool)


# speedup vs baseline: 1.2143x; 1.2143x over previous
import functools

import jax
import jax.numpy as jnp
from jax import lax
from jax.experimental import pallas as pl
from jax.experimental.pallas import tpu as pltpu
from jax.experimental.pallas import tpu_sc as plsc

_B, _N, _D, _L = 16, 2048, 512, 512
_NS = _N // 2
_K = min(int(2048 * 0.3), _NS)
_NNEW = _N - _K
_TOT = _B * _NNEW
_CH = 32
_NCHUNK = _TOT // _CH
_NW = 32
_CPW = -(-_NCHUNK // _NW)

def _metric_body(xfr_ref, me_ref, mo_ref):
    blk = xfr_ref[0]
    me_ref[0] = jnp.sum(blk[:, :_D], axis=1, keepdims=True)
    mo_ref[0] = jnp.sum(blk[:, _D:], axis=1, keepdims=True)


def _metric(xfr):
    return pl.pallas_call(
        _metric_body,
        grid=(_B,),
        in_specs=[pl.BlockSpec((1, _NS, 2 * _D), lambda b: (b, 0, 0))],
        out_specs=[pl.BlockSpec((1, _NS, 1), lambda b: (b, 0, 0)),
                   pl.BlockSpec((1, _NS, 1), lambda b: (b, 0, 0))],
        out_shape=[jax.ShapeDtypeStruct((_B, _NS, 1), jnp.float32),
                   jax.ShapeDtypeStruct((_B, _NS, 1), jnp.float32)],
    )(xfr)


def _plan_body(me_ref, mo_ref, ue_ref, uo_ref, pg_ref, mk_ref, cn_ref,
               sl_ref, dp_ref):
    b = pl.program_id(0)
    ns = _NS
    f32 = jnp.float32

    mean_e = me_ref[0] / f32(_D)
    mean_o = mo_ref[0] / f32(_D)
    norm2 = jnp.sum(mean_e * mean_e, keepdims=True) + \
        jnp.sum(mean_o * mean_o, keepdims=True)
    den = jnp.maximum(jnp.sqrt(norm2), f32(1e-12))
    a_col = mean_e / den
    b_col = mean_o / den

    iota_col = lax.broadcasted_iota(jnp.int32, (ns, 1), 0).astype(f32)
    iota_row = lax.broadcasted_iota(jnp.int32, (1, ns), 1).astype(f32)
    R = lax.broadcasted_iota(jnp.int32, (ns, ns), 0).astype(f32)
    C = lax.broadcasted_iota(jnp.int32, (ns, ns), 1).astype(f32)
    eye = (R == C).astype(f32)

    def to_row(v_col):
        return jnp.sum(eye * v_col, axis=0, keepdims=True)

    bmax = jnp.max(b_col, keepdims=True)
    bmin = jnp.min(b_col, keepdims=True)
    big = f32(ns + 1)
    jmax = jnp.min(jnp.where(b_col == bmax, iota_col, big), keepdims=True)
    jmin = jnp.min(jnp.where(b_col == bmin, iota_col, big), keepdims=True)

    zero = jnp.zeros((1, 1), f32)
    pos = a_col > 0
    neg = a_col < 0
    v_col = jnp.where(pos, a_col * bmax,
                      jnp.where(neg, a_col * bmin, zero))
    nidx_col = jnp.where(pos, jmax, jnp.where(neg, jmin, zero))

    v_row = to_row(v_col)
    gt = (v_row > v_col).astype(f32)
    eqlt = ((v_row == v_col) & (C < R)).astype(f32)
    rank_col = jnp.sum(gt + eqlt, axis=1, keepdims=True)
    sel = rank_col < f32(_K)
    sel_col = sel.astype(f32)

    sel_row = to_row(sel_col)
    tril = (C <= R).astype(f32)
    cum_incl = jnp.sum(tril * sel_row, axis=1, keepdims=True)
    cum_excl = cum_incl - sel_col

    def cum_at(t):
        return jnp.sum(jnp.where(iota_col == t, cum_incl, zero),
                       keepdims=True)

    cmax = cum_at(jmax)
    cmin = cum_at(jmin)
    c0 = cum_at(zero)
    cum_at_nidx = jnp.where(pos, cmax, jnp.where(neg, cmin, c0))

    i2_col = 2.0 * iota_col
    dst_col = 2.0 * nidx_col + 1.0
    ue = jnp.where(sel, dst_col - cum_at_nidx, i2_col - cum_excl)
    uo = i2_col + 1.0 - cum_incl
    ue_ref[0] = ue.astype(jnp.int32)
    uo_ref[0] = uo.astype(jnp.int32)

    rank_even = i2_col - cum_excl
    rank_odd = i2_col + 1.0 - cum_incl
    keep_col = 1.0 - sel_col
    C2 = lax.broadcasted_iota(jnp.int32, (ns, _NNEW), 1).astype(f32)
    e1t = (rank_even == C2).astype(f32) * keep_col
    e2t = (rank_odd == C2).astype(f32)
    p_row = jnp.sum(e1t * i2_col + e2t * (i2_col + 1.0),
                    axis=0, keepdims=True)
    pg_ref[0] = (p_row + f32(_N) * b.astype(f32)).astype(jnp.int32)

    t_list = [jmax, jmin, zero, zero]
    c_list = [cmax, cmin, c0, c0]
    masks = []
    cnts = []
    slots = []
    for t, ct in zip(t_list, c_list):
        m = sel_col * (nidx_col == t).astype(f32)
        masks.append(m)
        cnts.append(1.0 + jnp.sum(m, keepdims=True))
        slots.append(2.0 * t + 1.0 - ct + f32(_NNEW) * b.astype(f32))
    mk_ref[0] = jnp.concatenate(masks, axis=1)
    cn_ref[0] = jnp.concatenate([c.reshape(1, 1) for c in cnts], axis=0)
    sl_ref[0] = jnp.concatenate(slots, axis=1).astype(jnp.int32)
    dp_ref[0] = jnp.concatenate(t_list, axis=1).astype(jnp.int32)


def _plan(me, mo):
    return pl.pallas_call(
        _plan_body,
        grid=(_B,),
        in_specs=[pl.BlockSpec((1, _NS, 1), lambda b: (b, 0, 0)),
                  pl.BlockSpec((1, _NS, 1), lambda b: (b, 0, 0))],
        out_specs=[pl.BlockSpec((1, _NS, 1), lambda b: (b, 0, 0)),
                   pl.BlockSpec((1, _NS, 1), lambda b: (b, 0, 0)),
                   pl.BlockSpec((1, 1, _NNEW), lambda b: (b, 0, 0)),
                   pl.BlockSpec((1, _NS, 4), lambda b: (b, 0, 0)),
                   pl.BlockSpec((1, 4, 1), lambda b: (b, 0, 0)),
                   pl.BlockSpec((1, 1, 4), lambda b: (b, 0, 0)),
                   pl.BlockSpec((1, 1, 4), lambda b: (b, 0, 0))],
        out_shape=[jax.ShapeDtypeStruct((_B, _NS, 1), jnp.int32),
                   jax.ShapeDtypeStruct((_B, _NS, 1), jnp.int32),
                   jax.ShapeDtypeStruct((_B, 1, _NNEW), jnp.int32),
                   jax.ShapeDtypeStruct((_B, _NS, 4), jnp.float32),
                   jax.ShapeDtypeStruct((_B, 4, 1), jnp.float32),
                   jax.ShapeDtypeStruct((_B, 1, 4), jnp.int32),
                   jax.ShapeDtypeStruct((_B, 1, 4), jnp.int32)],
    )(me, mo)


def _special_body(dp_ref, xe_ref, re_ref, f0, f1, f2, f3, r0, r1, r2, r3,
                  mk_ref, cn_ref, sf_ref, sr_ref):
    mk = mk_ref[0]
    cnt = cn_ref[0] + jnp.float32(1e-6)

    def msum(x):
        return jnp.concatenate(
            [jnp.sum(x * mk[:, g:g + 1], axis=0, keepdims=True)
             for g in range(4)], axis=0)

    sfe = msum(xe_ref[0])
    sra = msum(re_ref[0])
    rows_f = jnp.concatenate(
        [f0[0, 0, 0], f1[0, 0, 0], f2[0, 0, 0], f3[0, 0, 0]], axis=0)
    rows_r = jnp.concatenate(
        [r0[0, 0, 0], r1[0, 0, 0], r2[0, 0, 0], r3[0, 0, 0]], axis=0)
    sf_ref[0] = (rows_f + sfe) / cnt
    sr_ref[0] = (rows_r + sra) / cnt


def _special(dpair, xfr, xrr, masks, cnts):
    xf5 = xfr.reshape(_B, _NS, 2, 1, _D)
    xr5 = xrr.reshape(_B, _NS, 2, 1, _L)

    def row_spec(j):
        return pl.BlockSpec(
            (1, 1, 1, 1, _D), lambda b, dp, j=j: (b, dp[b, 0, j], 1, 0, 0))

    grid_spec = pltpu.PrefetchScalarGridSpec(
        num_scalar_prefetch=1,
        grid=(_B,),
        in_specs=[pl.BlockSpec((1, _NS, _D), lambda b, dp: (b, 0, 0)),
                  pl.BlockSpec((1, _NS, _L), lambda b, dp: (b, 0, 0)),
                  row_spec(0), row_spec(1), row_spec(2), row_spec(3),
                  row_spec(0), row_spec(1), row_spec(2), row_spec(3),
                  pl.BlockSpec((1, _NS, 4), lambda b, dp: (b, 0, 0)),
                  pl.BlockSpec((1, 4, 1), lambda b, dp: (b, 0, 0))],
        out_specs=[pl.BlockSpec((1, 4, _D), lambda b, dp: (b, 0, 0)),
                   pl.BlockSpec((1, 4, _L), lambda b, dp: (b, 0, 0))],
    )
    return pl.pallas_call(
        _special_body,
        grid_spec=grid_spec,
        out_shape=[jax.ShapeDtypeStruct((_B, 4, _D), jnp.float32),
                   jax.ShapeDtypeStruct((_B, 4, _L), jnp.float32)],
    )(dpair, xfr, xrr, xf5, xf5, xf5, xf5, xr5, xr5, xr5, xr5, masks, cnts)


def _sc_gather(xf2d, xr2d, idx):
    mesh = plsc.VectorSubcoreMesh(core_axis_name="c", subcore_axis_name="s")

    @functools.partial(
        pl.kernel,
        mesh=mesh,
        out_type=(jax.ShapeDtypeStruct((_TOT, _D), jnp.float32),
                  jax.ShapeDtypeStruct((_TOT, _L), jnp.float32)),
        scratch_types=[pltpu.VMEM((_CH,), jnp.int32),
                       pltpu.VMEM((_CH, _D), jnp.float32),
                       pltpu.VMEM((_CH, _L), jnp.float32),
                       pltpu.SemaphoreType.DMA,
                       pltpu.SemaphoreType.DMA],
    )
    def gk(xf_hbm, xr_hbm, idx_hbm, of_hbm, or_hbm, idx_v, rf_v, rr_v,
           smf, smr):
        wid = lax.axis_index("s") * 2 + lax.axis_index("c")

        @pl.loop(0, _CPW)
        def _(j):
            c = wid * _CPW + j

            @pl.when(c < _NCHUNK)
            def _():
                base = c * _CH
                pltpu.sync_copy(idx_hbm.at[pl.ds(base, _CH)], idx_v)
                h1 = pltpu.async_copy(xf_hbm.at[idx_v], rf_v, smf)
                h2 = pltpu.async_copy(xr_hbm.at[idx_v], rr_v, smr)
                h1.wait()
                h2.wait()
                pltpu.sync_copy(rf_v, of_hbm.at[pl.ds(base, _CH)])
                pltpu.sync_copy(rr_v, or_hbm.at[pl.ds(base, _CH)])

    return gk(xf2d, xr2d, idx)


def _overwrite_body(sl_ref, gf_ref, gr_ref, sf_ref, sr_ref, of_ref, or_ref):
    del gf_ref, gr_ref
    of_ref[...] = sf_ref[...]
    or_ref[...] = sr_ref[...]


def _overwrite(slots_flat, gf3, gr3, sf3, sr3):
    dyn = pl.BlockSpec((1, 1, _D), lambda g, sl: (sl[g], 0, 0))
    anyspace = pl.BlockSpec(memory_space=pltpu.MemorySpace.HBM)
    grid_spec = pltpu.PrefetchScalarGridSpec(
        num_scalar_prefetch=1,
        grid=(_B * 4,),
        in_specs=[anyspace, anyspace,
                  pl.BlockSpec((1, 1, _D), lambda g, sl: (g, 0, 0)),
                  pl.BlockSpec((1, 1, _L), lambda g, sl: (g, 0, 0))],
        out_specs=[dyn, dyn],
    )
    return pl.pallas_call(
        _overwrite_body,
        grid_spec=grid_spec,
        out_shape=[jax.ShapeDtypeStruct((_TOT, 1, _D), jnp.float32),
                   jax.ShapeDtypeStruct((_TOT, 1, _L), jnp.float32)],
        input_output_aliases={1: 0, 2: 1},
    )(slots_flat, gf3, gr3, sf3, sr3)


@jax.jit
def kernel(x_feat, x_raw):
    xfr = x_feat.reshape(_B, _NS, 2 * _D)
    xrr = x_raw.reshape(_B, _NS, 2 * _L)

    me, mo = _metric(xfr)
    ue, uo, pglob, masks, cnts, slots, dpair = _plan(me, mo)
    unpool = jnp.concatenate([ue, uo], axis=2).reshape(_B, _N)

    sf, sr = _special(dpair, xfr, xrr, masks, cnts)

    gf, gr = _sc_gather(x_feat.reshape(_B * _N, _D),
                        x_raw.reshape(_B * _N, _L),
                        pglob.reshape(_TOT))

    of, orr = _overwrite(slots.reshape(_B * 4),
                         gf.reshape(_TOT, 1, _D),
                         gr.reshape(_TOT, 1, _L),
                         sf.reshape(_B * 4, 1, _D),
                         sr.reshape(_B * 4, 1, _L))

    return (of.reshape(_B, _NNEW, _D),
            orr.reshape(_B, _NNEW, _L),
            unpool)


# speedup vs baseline: 1.8922x; 1.5582x over previous
import functools

import jax
import jax.numpy as jnp
from jax import lax
from jax.experimental import pallas as pl
from jax.experimental.pallas import tpu as pltpu
from jax.experimental.pallas import tpu_sc as plsc

_B, _N, _D, _L = 16, 2048, 512, 512
_NS = _N // 2
_K = min(int(2048 * 0.3), _NS)
_NNEW = _N - _K
_TOT = _B * _NNEW
_CH = 32
_NCHUNK = _TOT // _CH
_NW = 32
_CPW = -(-_NCHUNK // _NW)

def _metric_body(xfr_ref, me_ref, mo_ref):
    blk = xfr_ref[0]
    me_ref[0] = jnp.sum(blk[:, :_D], axis=1, keepdims=True)
    mo_ref[0] = jnp.sum(blk[:, _D:], axis=1, keepdims=True)


def _metric(xfr):
    return pl.pallas_call(
        _metric_body,
        grid=(_B,),
        in_specs=[pl.BlockSpec((1, _NS, 2 * _D), lambda b: (b, 0, 0))],
        out_specs=[pl.BlockSpec((1, _NS, 1), lambda b: (b, 0, 0)),
                   pl.BlockSpec((1, _NS, 1), lambda b: (b, 0, 0))],
        out_shape=[jax.ShapeDtypeStruct((_B, _NS, 1), jnp.float32),
                   jax.ShapeDtypeStruct((_B, _NS, 1), jnp.float32)],
    )(xfr)


def _plan_body(me_ref, mo_ref, ue_ref, uo_ref, pg_ref, mk_ref, cn_ref,
               sl_ref, dp_ref):
    b = pl.program_id(0)
    ns = _NS
    f32 = jnp.float32

    mean_e = me_ref[0] / f32(_D)
    mean_o = mo_ref[0] / f32(_D)
    norm2 = jnp.sum(mean_e * mean_e, keepdims=True) + \
        jnp.sum(mean_o * mean_o, keepdims=True)
    den = jnp.maximum(jnp.sqrt(norm2), f32(1e-12))
    a_col = mean_e / den
    b_col = mean_o / den

    iota_col = lax.broadcasted_iota(jnp.int32, (ns, 1), 0).astype(f32)
    iota_row = lax.broadcasted_iota(jnp.int32, (1, ns), 1).astype(f32)
    R = lax.broadcasted_iota(jnp.int32, (ns, ns), 0).astype(f32)
    C = lax.broadcasted_iota(jnp.int32, (ns, ns), 1).astype(f32)
    eye = (R == C).astype(f32)

    def to_row(v_col):
        return jnp.sum(eye * v_col, axis=0, keepdims=True)

    bmax = jnp.max(b_col, keepdims=True)
    bmin = jnp.min(b_col, keepdims=True)
    big = f32(ns + 1)
    jmax = jnp.min(jnp.where(b_col == bmax, iota_col, big), keepdims=True)
    jmin = jnp.min(jnp.where(b_col == bmin, iota_col, big), keepdims=True)

    zero = jnp.zeros((1, 1), f32)
    pos = a_col > 0
    neg = a_col < 0
    v_col = jnp.where(pos, a_col * bmax,
                      jnp.where(neg, a_col * bmin, zero))
    nidx_col = jnp.where(pos, jmax, jnp.where(neg, jmin, zero))

    v_row = to_row(v_col)
    gt = (v_row > v_col).astype(f32)
    eqlt = ((v_row == v_col) & (C < R)).astype(f32)
    rank_col = jnp.sum(gt + eqlt, axis=1, keepdims=True)
    sel = rank_col < f32(_K)
    sel_col = sel.astype(f32)

    sel_row = to_row(sel_col)
    tril = (C <= R).astype(f32)
    cum_incl = jnp.sum(tril * sel_row, axis=1, keepdims=True)
    cum_excl = cum_incl - sel_col

    def cum_at(t):
        return jnp.sum(jnp.where(iota_col == t, cum_incl, zero),
                       keepdims=True)

    cmax = cum_at(jmax)
    cmin = cum_at(jmin)
    c0 = cum_at(zero)
    cum_at_nidx = jnp.where(pos, cmax, jnp.where(neg, cmin, c0))

    i2_col = 2.0 * iota_col
    dst_col = 2.0 * nidx_col + 1.0
    ue = jnp.where(sel, dst_col - cum_at_nidx, i2_col - cum_excl)
    uo = i2_col + 1.0 - cum_incl
    ue_ref[0] = ue.astype(jnp.int32)
    uo_ref[0] = uo.astype(jnp.int32)

    rank_even = i2_col - cum_excl
    rank_odd = i2_col + 1.0 - cum_incl
    keep_col = 1.0 - sel_col
    C2 = lax.broadcasted_iota(jnp.int32, (ns, _NNEW), 1).astype(f32)
    e1t = (rank_even == C2).astype(f32) * keep_col
    e2t = (rank_odd == C2).astype(f32)
    p_row = jnp.sum(e1t * i2_col + e2t * (i2_col + 1.0),
                    axis=0, keepdims=True)
    pg_ref[0] = (p_row + f32(_N) * b.astype(f32)).astype(jnp.int32)

    t_list = [jmax, jmin, zero, zero]
    c_list = [cmax, cmin, c0, c0]
    masks = []
    cnts = []
    slots = []
    for t, ct in zip(t_list, c_list):
        m = sel_col * (nidx_col == t).astype(f32)
        masks.append(m)
        cnts.append(1.0 + jnp.sum(m, keepdims=True))
        slots.append(2.0 * t + 1.0 - ct + f32(_NNEW) * b.astype(f32))
    mk_ref[0] = jnp.concatenate(masks, axis=1)
    cn_ref[0] = jnp.concatenate([c.reshape(1, 1) for c in cnts], axis=0)
    sl_ref[0] = jnp.concatenate(slots, axis=1).astype(jnp.int32)
    dp_ref[0] = jnp.concatenate(t_list, axis=1).astype(jnp.int32)


def _plan(me, mo):
    return pl.pallas_call(
        _plan_body,
        grid=(_B,),
        in_specs=[pl.BlockSpec((1, _NS, 1), lambda b: (b, 0, 0)),
                  pl.BlockSpec((1, _NS, 1), lambda b: (b, 0, 0))],
        out_specs=[pl.BlockSpec((1, _NS, 1), lambda b: (b, 0, 0)),
                   pl.BlockSpec((1, _NS, 1), lambda b: (b, 0, 0)),
                   pl.BlockSpec((1, 1, _NNEW), lambda b: (b, 0, 0)),
                   pl.BlockSpec((1, _NS, 4), lambda b: (b, 0, 0)),
                   pl.BlockSpec((1, 4, 1), lambda b: (b, 0, 0)),
                   pl.BlockSpec((1, 1, 4), lambda b: (b, 0, 0)),
                   pl.BlockSpec((1, 1, 4), lambda b: (b, 0, 0))],
        out_shape=[jax.ShapeDtypeStruct((_B, _NS, 1), jnp.int32),
                   jax.ShapeDtypeStruct((_B, _NS, 1), jnp.int32),
                   jax.ShapeDtypeStruct((_B, 1, _NNEW), jnp.int32),
                   jax.ShapeDtypeStruct((_B, _NS, 4), jnp.float32),
                   jax.ShapeDtypeStruct((_B, 4, 1), jnp.float32),
                   jax.ShapeDtypeStruct((_B, 1, 4), jnp.int32),
                   jax.ShapeDtypeStruct((_B, 1, 4), jnp.int32)],
    )(me, mo)


def _special_body(dp_ref, xe_ref, re_ref, f0, f1, f2, f3, r0, r1, r2, r3,
                  mk_ref, cn_ref, sf_ref, sr_ref):
    mk = mk_ref[0]
    cnt = cn_ref[0] + jnp.float32(1e-6)

    def msum(x):
        return jnp.concatenate(
            [jnp.sum(x * mk[:, g:g + 1], axis=0, keepdims=True)
             for g in range(4)], axis=0)

    sfe = msum(xe_ref[0])
    sra = msum(re_ref[0])
    rows_f = jnp.concatenate(
        [f0[0, 0, 0], f1[0, 0, 0], f2[0, 0, 0], f3[0, 0, 0]], axis=0)
    rows_r = jnp.concatenate(
        [r0[0, 0, 0], r1[0, 0, 0], r2[0, 0, 0], r3[0, 0, 0]], axis=0)
    sf_ref[0] = (rows_f + sfe) / cnt
    sr_ref[0] = (rows_r + sra) / cnt


def _special(dpair, xfr, xrr, masks, cnts):
    xf5 = xfr.reshape(_B, _NS, 2, 1, _D)
    xr5 = xrr.reshape(_B, _NS, 2, 1, _L)

    def row_spec(j):
        return pl.BlockSpec(
            (1, 1, 1, 1, _D), lambda b, dp, j=j: (b, dp[b, 0, j], 1, 0, 0))

    grid_spec = pltpu.PrefetchScalarGridSpec(
        num_scalar_prefetch=1,
        grid=(_B,),
        in_specs=[pl.BlockSpec((1, _NS, _D), lambda b, dp: (b, 0, 0)),
                  pl.BlockSpec((1, _NS, _L), lambda b, dp: (b, 0, 0)),
                  row_spec(0), row_spec(1), row_spec(2), row_spec(3),
                  row_spec(0), row_spec(1), row_spec(2), row_spec(3),
                  pl.BlockSpec((1, _NS, 4), lambda b, dp: (b, 0, 0)),
                  pl.BlockSpec((1, 4, 1), lambda b, dp: (b, 0, 0))],
        out_specs=[pl.BlockSpec((1, 4, _D), lambda b, dp: (b, 0, 0)),
                   pl.BlockSpec((1, 4, _L), lambda b, dp: (b, 0, 0))],
    )
    return pl.pallas_call(
        _special_body,
        grid_spec=grid_spec,
        out_shape=[jax.ShapeDtypeStruct((_B, 4, _D), jnp.float32),
                   jax.ShapeDtypeStruct((_B, 4, _L), jnp.float32)],
    )(dpair, xfr, xrr, xf5, xf5, xf5, xf5, xr5, xr5, xr5, xr5, masks, cnts)


def _sc_gather(xf2d, xr2d, idx):
    mesh = plsc.VectorSubcoreMesh(core_axis_name="c", subcore_axis_name="s")

    @functools.partial(
        pl.kernel,
        mesh=mesh,
        out_type=(jax.ShapeDtypeStruct((_TOT, _D), jnp.float32),
                  jax.ShapeDtypeStruct((_TOT, _L), jnp.float32)),
        scratch_types=[pltpu.VMEM((_CH,), jnp.int32),
                       pltpu.VMEM((_CH, _D), jnp.float32),
                       pltpu.VMEM((_CH, _L), jnp.float32),
                       pltpu.SemaphoreType.DMA,
                       pltpu.SemaphoreType.DMA],
    )
    def gk(xf_hbm, xr_hbm, idx_hbm, of_hbm, or_hbm, idx_v, rf_v, rr_v,
           smf, smr):
        wid = lax.axis_index("s") * 2 + lax.axis_index("c")

        @pl.loop(0, _CPW)
        def _(j):
            c = wid * _CPW + j

            @pl.when(c < _NCHUNK)
            def _():
                base = c * _CH
                pltpu.sync_copy(idx_hbm.at[pl.ds(base, _CH)], idx_v)
                h1 = pltpu.async_copy(xf_hbm.at[idx_v], rf_v, smf)
                h2 = pltpu.async_copy(xr_hbm.at[idx_v], rr_v, smr)
                h1.wait()
                h2.wait()
                pltpu.sync_copy(rf_v, of_hbm.at[pl.ds(base, _CH)])
                pltpu.sync_copy(rr_v, or_hbm.at[pl.ds(base, _CH)])

    return gk(xf2d, xr2d, idx)


def _overwrite_body(sl_ref, gf_ref, gr_ref, sf_ref, sr_ref, of_ref, or_ref,
                    semf, semr):
    del gf_ref, gr_ref

    def fire(j, _):
        s = sl_ref[j]
        pltpu.make_async_copy(sf_ref.at[j], of_ref.at[s], semf).start()
        pltpu.make_async_copy(sr_ref.at[j], or_ref.at[s], semr).start()
        return 0

    jax.lax.fori_loop(0, _B * 4, fire, 0)

    def drain(j, _):
        pltpu.make_async_copy(sf_ref.at[0], of_ref.at[0], semf).wait()
        pltpu.make_async_copy(sr_ref.at[0], or_ref.at[0], semr).wait()
        return 0

    jax.lax.fori_loop(0, _B * 4, drain, 0)


def _overwrite(slots_flat, gf, gr, sf, sr):
    anyspace = pl.BlockSpec(memory_space=pltpu.MemorySpace.HBM)
    grid_spec = pltpu.PrefetchScalarGridSpec(
        num_scalar_prefetch=1,
        grid=(1,),
        in_specs=[anyspace, anyspace,
                  pl.BlockSpec((_B * 4, _D), lambda g, sl: (0, 0)),
                  pl.BlockSpec((_B * 4, _L), lambda g, sl: (0, 0))],
        out_specs=[anyspace, anyspace],
        scratch_shapes=[pltpu.SemaphoreType.DMA, pltpu.SemaphoreType.DMA],
    )
    return pl.pallas_call(
        _overwrite_body,
        grid_spec=grid_spec,
        out_shape=[jax.ShapeDtypeStruct((_TOT, _D), jnp.float32),
                   jax.ShapeDtypeStruct((_TOT, _L), jnp.float32)],
        input_output_aliases={1: 0, 2: 1},
    )(slots_flat, gf, gr, sf, sr)


@jax.jit
def kernel(x_feat, x_raw):
    xfr = x_feat.reshape(_B, _NS, 2 * _D)
    xrr = x_raw.reshape(_B, _NS, 2 * _L)

    me, mo = _metric(xfr)
    ue, uo, pglob, masks, cnts, slots, dpair = _plan(me, mo)
    unpool = jnp.concatenate([ue, uo], axis=2).reshape(_B, _N)

    sf, sr = _special(dpair, xfr, xrr, masks, cnts)

    gf, gr = _sc_gather(x_feat.reshape(_B * _N, _D),
                        x_raw.reshape(_B * _N, _L),
                        pglob.reshape(_TOT))

    of, orr = _overwrite(slots.reshape(_B * 4), gf, gr,
                         sf.reshape(_B * 4, _D), sr.reshape(_B * 4, _L))

    return (of.reshape(_B, _NNEW, _D),
            orr.reshape(_B, _NNEW, _L),
            unpool)


# speedup vs baseline: 3.6153x; 1.9107x over previous
import functools

import jax
import jax.numpy as jnp
from jax import lax
from jax.experimental import pallas as pl
from jax.experimental.pallas import tpu as pltpu
from jax.experimental.pallas import tpu_sc as plsc

_B, _N, _D, _L = 16, 2048, 512, 512
_NS = _N // 2
_K = min(int(2048 * 0.3), _NS)
_NNEW = _N - _K
_TOT = _B * _NNEW
_CH = 32
_NCHUNK = _TOT // _CH
_NW = 32
_CPW = -(-_NCHUNK // _NW)

def _metric_body(xfr_ref, me_ref, mo_ref):
    blk = xfr_ref[0]
    me_ref[0] = jnp.sum(blk[:, :_D], axis=1, keepdims=True)
    mo_ref[0] = jnp.sum(blk[:, _D:], axis=1, keepdims=True)


def _metric(xfr):
    return pl.pallas_call(
        _metric_body,
        grid=(_B,),
        in_specs=[pl.BlockSpec((1, _NS, 2 * _D), lambda b: (b, 0, 0))],
        out_specs=[pl.BlockSpec((1, _NS, 1), lambda b: (b, 0, 0)),
                   pl.BlockSpec((1, _NS, 1), lambda b: (b, 0, 0))],
        out_shape=[jax.ShapeDtypeStruct((_B, _NS, 1), jnp.float32),
                   jax.ShapeDtypeStruct((_B, _NS, 1), jnp.float32)],
    )(xfr)


def _plan_body(me_ref, mo_ref, ue_ref, uo_ref, pg_ref, mk_ref, cn_ref,
               sl_ref, dp_ref):
    b = pl.program_id(0)
    ns = _NS
    f32 = jnp.float32

    mean_e = me_ref[0] / f32(_D)
    mean_o = mo_ref[0] / f32(_D)
    norm2 = jnp.sum(mean_e * mean_e, keepdims=True) + \
        jnp.sum(mean_o * mean_o, keepdims=True)
    den = jnp.maximum(jnp.sqrt(norm2), f32(1e-12))
    a_col = mean_e / den
    b_col = mean_o / den

    iota_col = lax.broadcasted_iota(jnp.int32, (ns, 1), 0).astype(f32)
    iota_row = lax.broadcasted_iota(jnp.int32, (1, ns), 1).astype(f32)
    R = lax.broadcasted_iota(jnp.int32, (ns, ns), 0).astype(f32)
    C = lax.broadcasted_iota(jnp.int32, (ns, ns), 1).astype(f32)
    eye = (R == C).astype(f32)

    def to_row(v_col):
        return jnp.sum(eye * v_col, axis=0, keepdims=True)

    bmax = jnp.max(b_col, keepdims=True)
    bmin = jnp.min(b_col, keepdims=True)
    big = f32(ns + 1)
    jmax = jnp.min(jnp.where(b_col == bmax, iota_col, big), keepdims=True)
    jmin = jnp.min(jnp.where(b_col == bmin, iota_col, big), keepdims=True)

    zero = jnp.zeros((1, 1), f32)
    pos = a_col > 0
    neg = a_col < 0
    v_col = jnp.where(pos, a_col * bmax,
                      jnp.where(neg, a_col * bmin, zero))
    nidx_col = jnp.where(pos, jmax, jnp.where(neg, jmin, zero))

    v_row = to_row(v_col)
    gt = (v_row > v_col).astype(f32)
    eqlt = ((v_row == v_col) & (C < R)).astype(f32)
    rank_col = jnp.sum(gt + eqlt, axis=1, keepdims=True)
    sel = rank_col < f32(_K)
    sel_col = sel.astype(f32)

    sel_row = to_row(sel_col)
    tril = (C <= R).astype(f32)
    cum_incl = jnp.sum(tril * sel_row, axis=1, keepdims=True)
    cum_excl = cum_incl - sel_col

    def cum_at(t):
        return jnp.sum(jnp.where(iota_col == t, cum_incl, zero),
                       keepdims=True)

    cmax = cum_at(jmax)
    cmin = cum_at(jmin)
    c0 = cum_at(zero)
    cum_at_nidx = jnp.where(pos, cmax, jnp.where(neg, cmin, c0))

    i2_col = 2.0 * iota_col
    dst_col = 2.0 * nidx_col + 1.0
    ue = jnp.where(sel, dst_col - cum_at_nidx, i2_col - cum_excl)
    uo = i2_col + 1.0 - cum_incl
    ue_ref[0] = ue.astype(jnp.int32)
    uo_ref[0] = uo.astype(jnp.int32)

    rank_even = i2_col - cum_excl
    rank_odd = i2_col + 1.0 - cum_incl
    keep_col = 1.0 - sel_col
    C2 = lax.broadcasted_iota(jnp.int32, (ns, _NNEW), 1).astype(f32)
    e1t = (rank_even == C2).astype(f32) * keep_col
    e2t = (rank_odd == C2).astype(f32)
    p_row = jnp.sum(e1t * i2_col + e2t * (i2_col + 1.0),
                    axis=0, keepdims=True)
    pg_ref[0] = (p_row + f32(_N) * b.astype(f32)).astype(jnp.int32)

    t_list = [jmax, jmin, zero, zero]
    c_list = [cmax, cmin, c0, c0]
    masks = []
    cnts = []
    slots = []
    for t, ct in zip(t_list, c_list):
        m = sel_col * (nidx_col == t).astype(f32)
        masks.append(m)
        cnts.append(1.0 + jnp.sum(m, keepdims=True))
        slots.append(2.0 * t + 1.0 - ct)
    mk_ref[0] = jnp.concatenate(masks, axis=1)
    cn_ref[0] = jnp.concatenate([c.reshape(1, 1) for c in cnts], axis=0)
    sl_ref[0] = jnp.concatenate(slots, axis=1).astype(jnp.int32)
    bofs = f32(_N) * b.astype(f32)
    dp_ref[0] = jnp.concatenate(
        [2.0 * t + 1.0 + bofs for t in t_list], axis=1).astype(jnp.int32)


def _plan(me, mo):
    return pl.pallas_call(
        _plan_body,
        grid=(_B,),
        in_specs=[pl.BlockSpec((1, _NS, 1), lambda b: (b, 0, 0)),
                  pl.BlockSpec((1, _NS, 1), lambda b: (b, 0, 0))],
        out_specs=[pl.BlockSpec((1, _NS, 1), lambda b: (b, 0, 0)),
                   pl.BlockSpec((1, _NS, 1), lambda b: (b, 0, 0)),
                   pl.BlockSpec((1, 1, _NNEW), lambda b: (b, 0, 0)),
                   pl.BlockSpec((1, _NS, 4), lambda b: (b, 0, 0)),
                   pl.BlockSpec((1, 4, 1), lambda b: (b, 0, 0)),
                   pl.BlockSpec((1, 1, 4), lambda b: (b, 0, 0)),
                   pl.BlockSpec((1, 1, 4), lambda b: (b, 0, 0))],
        out_shape=[jax.ShapeDtypeStruct((_B, _NS, 1), jnp.int32),
                   jax.ShapeDtypeStruct((_B, _NS, 1), jnp.int32),
                   jax.ShapeDtypeStruct((_B, 1, _NNEW), jnp.int32),
                   jax.ShapeDtypeStruct((_B, _NS, 4), jnp.float32),
                   jax.ShapeDtypeStruct((_B, 4, 1), jnp.float32),
                   jax.ShapeDtypeStruct((_B, 1, 4), jnp.int32),
                   jax.ShapeDtypeStruct((_B, 1, 4), jnp.int32)],
    )(me, mo)


def _special_body(xe_ref, re_ref, mk_ref, cn_ref, sf_ref, sr_ref, iv_ref):
    mk = mk_ref[0]
    inv = 1.0 / (cn_ref[0] + jnp.float32(1e-6))

    def msum(x):
        return jnp.concatenate(
            [jnp.sum(x * mk[:, g:g + 1], axis=0, keepdims=True)
             for g in range(4)], axis=0)

    sf_ref[0] = msum(xe_ref[0]) * inv
    sr_ref[0] = msum(re_ref[0]) * inv
    iv_ref[0] = inv


def _special(xfr, xrr, masks, cnts):
    return pl.pallas_call(
        _special_body,
        grid=(_B,),
        in_specs=[pl.BlockSpec((1, _NS, _D), lambda b: (b, 0, 0)),
                  pl.BlockSpec((1, _NS, _L), lambda b: (b, 0, 0)),
                  pl.BlockSpec((1, _NS, 4), lambda b: (b, 0, 0)),
                  pl.BlockSpec((1, 4, 1), lambda b: (b, 0, 0))],
        out_specs=[pl.BlockSpec((1, 4, _D), lambda b: (b, 0, 0)),
                   pl.BlockSpec((1, 4, _L), lambda b: (b, 0, 0)),
                   pl.BlockSpec((1, 4, 1), lambda b: (b, 0, 0))],
        out_shape=[jax.ShapeDtypeStruct((_B, 4, _D), jnp.float32),
                   jax.ShapeDtypeStruct((_B, 4, _L), jnp.float32),
                   jax.ShapeDtypeStruct((_B, 4, 1), jnp.float32)],
    )(xfr, xrr, masks, cnts)


def _sc_gather(xf2d, xr2d, idx2d):
    mesh = plsc.VectorSubcoreMesh(core_axis_name="c", subcore_axis_name="s")

    @functools.partial(
        pl.kernel,
        mesh=mesh,
        out_type=(jax.ShapeDtypeStruct((_B, _NNEW, _D), jnp.float32),
                  jax.ShapeDtypeStruct((_B, _NNEW, _L), jnp.float32)),
        scratch_types=[pltpu.VMEM((32,), jnp.int32),
                       pltpu.VMEM((32, _D), jnp.float32),
                       pltpu.VMEM((32, _L), jnp.float32),
                       pltpu.SemaphoreType.DMA,
                       pltpu.SemaphoreType.DMA],
    )
    def gk(xf_hbm, xr_hbm, idx_hbm, of_hbm, or_hbm,
           idx_v, rf_v, rr_v, smf, smr):
        wid = lax.axis_index("s") * 2 + lax.axis_index("c")
        b = wid // 2
        half = wid - 2 * b

        @pl.loop(0, 23)
        def _(j):
            c = half + 2 * j

            @pl.when(c < 45)
            def _():
                base = c * 32
                pltpu.sync_copy(idx_hbm.at[pl.ds(b * 1440 + base, 32)],
                                idx_v)
                h1 = pltpu.async_copy(xf_hbm.at[idx_v], rf_v, smf)
                h2 = pltpu.async_copy(xr_hbm.at[idx_v], rr_v, smr)
                h1.wait()
                h2.wait()

                @pl.when(c < 44)
                def _():
                    pltpu.sync_copy(rf_v, of_hbm.at[b].at[pl.ds(base, 32)])
                    pltpu.sync_copy(rr_v, or_hbm.at[b].at[pl.ds(base, 32)])

                @pl.when(c == 44)
                def _():
                    @pl.loop(0, 26)
                    def _(r):
                        pltpu.sync_copy(rf_v.at[r], of_hbm.at[b, 1408 + r])
                        pltpu.sync_copy(rr_v.at[r], or_hbm.at[b, 1408 + r])

    return gk(xf2d, xr2d, idx2d.reshape(_B * 1440))


def _overwrite_body(sl_ref, dg_ref, gf_ref, gr_ref, xf_ref, xr_ref,
                    sdf_ref, sdr_ref, iv_ref, of_ref, or_ref,
                    xfs, xrs, ofs, ors, sma, smb, smc, smd):
    del gf_ref, gr_ref
    n = _B * 4

    def fire_in(g, _):
        d = dg_ref[g]
        pltpu.make_async_copy(xf_ref.at[d], xfs.at[g], sma).start()
        pltpu.make_async_copy(xr_ref.at[d], xrs.at[g], smb).start()
        return 0

    jax.lax.fori_loop(0, n, fire_in, 0)

    def drain_in(g, _):
        pltpu.make_async_copy(xf_ref.at[0], xfs.at[0], sma).wait()
        pltpu.make_async_copy(xr_ref.at[0], xrs.at[0], smb).wait()
        return 0

    jax.lax.fori_loop(0, n, drain_in, 0)

    ofs[...] = xfs[...] * iv_ref[...] + sdf_ref[...]
    ors[...] = xrs[...] * iv_ref[...] + sdr_ref[...]

    def fire_out(g, _):
        b = g // 4
        s = sl_ref[g]
        pltpu.make_async_copy(ofs.at[g], of_ref.at[b, s], smc).start()
        pltpu.make_async_copy(ors.at[g], or_ref.at[b, s], smd).start()
        return 0

    jax.lax.fori_loop(0, n, fire_out, 0)

    def drain_out(g, _):
        pltpu.make_async_copy(ofs.at[0], of_ref.at[0, 0], smc).wait()
        pltpu.make_async_copy(ors.at[0], or_ref.at[0, 0], smd).wait()
        return 0

    jax.lax.fori_loop(0, n, drain_out, 0)


def _overwrite(slots_flat, dstg_flat, gf, gr, xf2d, xr2d, sdf, sdr, inv):
    anyspace = pl.BlockSpec(memory_space=pltpu.MemorySpace.HBM)
    grid_spec = pltpu.PrefetchScalarGridSpec(
        num_scalar_prefetch=2,
        grid=(1,),
        in_specs=[anyspace, anyspace, anyspace, anyspace,
                  pl.BlockSpec((_B * 4, _D), lambda g, sl, dg: (0, 0)),
                  pl.BlockSpec((_B * 4, _L), lambda g, sl, dg: (0, 0)),
                  pl.BlockSpec((_B * 4, 1), lambda g, sl, dg: (0, 0))],
        out_specs=[anyspace, anyspace],
        scratch_shapes=[pltpu.VMEM((_B * 4, _D), jnp.float32),
                        pltpu.VMEM((_B * 4, _L), jnp.float32),
                        pltpu.VMEM((_B * 4, _D), jnp.float32),
                        pltpu.VMEM((_B * 4, _L), jnp.float32),
                        pltpu.SemaphoreType.DMA, pltpu.SemaphoreType.DMA,
                        pltpu.SemaphoreType.DMA, pltpu.SemaphoreType.DMA],
    )
    return pl.pallas_call(
        _overwrite_body,
        grid_spec=grid_spec,
        out_shape=[jax.ShapeDtypeStruct((_B, _NNEW, _D), jnp.float32),
                   jax.ShapeDtypeStruct((_B, _NNEW, _L), jnp.float32)],
        input_output_aliases={2: 0, 3: 1},
    )(slots_flat, dstg_flat, gf, gr, xf2d, xr2d, sdf, sdr, inv)


@jax.jit
def kernel(x_feat, x_raw):
    xfr = x_feat.reshape(_B, _NS, 2 * _D)
    xrr = x_raw.reshape(_B, _NS, 2 * _L)

    me, mo = _metric(xfr)
    ue, uo, pglob, masks, cnts, slots, dstg = _plan(me, mo)
    unpool = jnp.concatenate([ue, uo], axis=2).reshape(_B, _N)

    sdf, sdr, inv = _special(xfr, xrr, masks, cnts)

    idx2d = jnp.pad(pglob.reshape(_B, _NNEW), ((0, 0), (0, 1440 - _NNEW)))
    gf, gr = _sc_gather(x_feat.reshape(_B * _N, _D),
                        x_raw.reshape(_B * _N, _L),
                        idx2d)

    of, orr = _overwrite(slots.reshape(_B * 4), dstg.reshape(_B * 4),
                         gf, gr,
                         x_feat.reshape(_B * _N, _D),
                         x_raw.reshape(_B * _N, _L),
                         sdf.reshape(_B * 4, _D), sdr.reshape(_B * 4, _L),
                         inv.reshape(_B * 4, 1))

    return (of, orr, unpool)


# speedup vs baseline: 3.6185x; 1.0009x over previous
import functools

import jax
import jax.numpy as jnp
from jax import lax
from jax.experimental import pallas as pl
from jax.experimental.pallas import tpu as pltpu
from jax.experimental.pallas import tpu_sc as plsc

_B, _N, _D, _L = 16, 2048, 512, 512
_NS = _N // 2
_K = min(int(2048 * 0.3), _NS)
_NNEW = _N - _K

def _metric_body(xfr_ref, me_ref, mo_ref):
    blk = xfr_ref[0]
    me_ref[0] = jnp.sum(blk[:, :_D], axis=1, keepdims=True)
    mo_ref[0] = jnp.sum(blk[:, _D:], axis=1, keepdims=True)


def _metric(xfr):
    return pl.pallas_call(
        _metric_body,
        grid=(_B,),
        in_specs=[pl.BlockSpec((1, _NS, 2 * _D), lambda b: (b, 0, 0))],
        out_specs=[pl.BlockSpec((1, _NS, 1), lambda b: (b, 0, 0)),
                   pl.BlockSpec((1, _NS, 1), lambda b: (b, 0, 0))],
        out_shape=[jax.ShapeDtypeStruct((_B, _NS, 1), jnp.float32),
                   jax.ShapeDtypeStruct((_B, _NS, 1), jnp.float32)],
    )(xfr)


def _plan_body(me_ref, mo_ref, ue_ref, uo_ref, pg_ref, mk_ref, cn_ref,
               sl_ref, dp_ref):
    b = pl.program_id(0)
    ns = _NS
    f32 = jnp.float32

    mean_e = me_ref[0] / f32(_D)
    mean_o = mo_ref[0] / f32(_D)
    norm2 = jnp.sum(mean_e * mean_e, keepdims=True) + \
        jnp.sum(mean_o * mean_o, keepdims=True)
    den = jnp.maximum(jnp.sqrt(norm2), f32(1e-12))
    a_col = mean_e / den
    b_col = mean_o / den

    iota_col = lax.broadcasted_iota(jnp.int32, (ns, 1), 0).astype(f32)
    iota_row = lax.broadcasted_iota(jnp.int32, (1, ns), 1).astype(f32)
    R = lax.broadcasted_iota(jnp.int32, (ns, ns), 0).astype(f32)
    C = lax.broadcasted_iota(jnp.int32, (ns, ns), 1).astype(f32)
    eye = (R == C).astype(f32)

    def to_row(v_col):
        return jnp.sum(eye * v_col, axis=0, keepdims=True)

    bmax = jnp.max(b_col, keepdims=True)
    bmin = jnp.min(b_col, keepdims=True)
    big = f32(ns + 1)
    jmax = jnp.min(jnp.where(b_col == bmax, iota_col, big), keepdims=True)
    jmin = jnp.min(jnp.where(b_col == bmin, iota_col, big), keepdims=True)

    zero = jnp.zeros((1, 1), f32)
    pos = a_col > 0
    neg = a_col < 0
    v_col = jnp.where(pos, a_col * bmax,
                      jnp.where(neg, a_col * bmin, zero))
    nidx_col = jnp.where(pos, jmax, jnp.where(neg, jmin, zero))

    v_row = to_row(v_col)
    gt = (v_row > v_col).astype(f32)
    eqlt = ((v_row == v_col) & (C < R)).astype(f32)
    rank_col = jnp.sum(gt + eqlt, axis=1, keepdims=True)
    sel = rank_col < f32(_K)
    sel_col = sel.astype(f32)

    sel_row = to_row(sel_col)
    tril = (C <= R).astype(f32)
    cum_incl = jnp.sum(tril * sel_row, axis=1, keepdims=True)
    cum_excl = cum_incl - sel_col

    def cum_at(t):
        return jnp.sum(jnp.where(iota_col == t, cum_incl, zero),
                       keepdims=True)

    cmax = cum_at(jmax)
    cmin = cum_at(jmin)
    c0 = cum_at(zero)
    cum_at_nidx = jnp.where(pos, cmax, jnp.where(neg, cmin, c0))

    i2_col = 2.0 * iota_col
    dst_col = 2.0 * nidx_col + 1.0
    ue = jnp.where(sel, dst_col - cum_at_nidx, i2_col - cum_excl)
    uo = i2_col + 1.0 - cum_incl
    ue_ref[0] = ue.astype(jnp.int32)
    uo_ref[0] = uo.astype(jnp.int32)

    rank_even = i2_col - cum_excl
    rank_odd = i2_col + 1.0 - cum_incl
    keep_col = 1.0 - sel_col
    C2 = lax.broadcasted_iota(jnp.int32, (ns, _NNEW), 1).astype(f32)
    e1t = (rank_even == C2).astype(f32) * keep_col
    e2t = (rank_odd == C2).astype(f32)
    p_row = jnp.sum(e1t * i2_col + e2t * (i2_col + 1.0),
                    axis=0, keepdims=True)
    pg_ref[0] = (p_row + f32(_N) * b.astype(f32)).astype(jnp.int32)

    t_list = [jmax, jmin, zero, zero]
    c_list = [cmax, cmin, c0, c0]
    masks = []
    cnts = []
    slots = []
    for t, ct in zip(t_list, c_list):
        m = sel_col * (nidx_col == t).astype(f32)
        masks.append(m)
        cnts.append(1.0 + jnp.sum(m, keepdims=True))
        slots.append(2.0 * t + 1.0 - ct)
    mk_ref[0] = jnp.concatenate(masks, axis=1)
    cn_ref[0] = jnp.concatenate([c.reshape(1, 1) for c in cnts], axis=0)
    sl_ref[0] = jnp.concatenate(slots, axis=1).astype(jnp.int32)
    bofs = f32(_N) * b.astype(f32)
    dp_ref[0] = jnp.concatenate(
        [2.0 * t + 1.0 + bofs for t in t_list], axis=1).astype(jnp.int32)


def _plan(me, mo):
    return pl.pallas_call(
        _plan_body,
        grid=(_B,),
        in_specs=[pl.BlockSpec((1, _NS, 1), lambda b: (b, 0, 0)),
                  pl.BlockSpec((1, _NS, 1), lambda b: (b, 0, 0))],
        out_specs=[pl.BlockSpec((1, _NS, 1), lambda b: (b, 0, 0)),
                   pl.BlockSpec((1, _NS, 1), lambda b: (b, 0, 0)),
                   pl.BlockSpec((1, 1, _NNEW), lambda b: (b, 0, 0)),
                   pl.BlockSpec((1, _NS, 4), lambda b: (b, 0, 0)),
                   pl.BlockSpec((1, 4, 1), lambda b: (b, 0, 0)),
                   pl.BlockSpec((1, 1, 4), lambda b: (b, 0, 0)),
                   pl.BlockSpec((1, 1, 4), lambda b: (b, 0, 0))],
        out_shape=[jax.ShapeDtypeStruct((_B, _NS, 1), jnp.int32),
                   jax.ShapeDtypeStruct((_B, _NS, 1), jnp.int32),
                   jax.ShapeDtypeStruct((_B, 1, _NNEW), jnp.int32),
                   jax.ShapeDtypeStruct((_B, _NS, 4), jnp.float32),
                   jax.ShapeDtypeStruct((_B, 4, 1), jnp.float32),
                   jax.ShapeDtypeStruct((_B, 1, 4), jnp.int32),
                   jax.ShapeDtypeStruct((_B, 1, 4), jnp.int32)],
    )(me, mo)


def _special_body(xe_ref, re_ref, mk_ref, cn_ref, sf_ref, sr_ref, iv_ref):
    mk = mk_ref[0]
    inv = 1.0 / (cn_ref[0] + jnp.float32(1e-6))

    def msum(x):
        return jnp.concatenate(
            [jnp.sum(x * mk[:, g:g + 1], axis=0, keepdims=True)
             for g in range(4)], axis=0)

    sf_ref[0] = msum(xe_ref[0]) * inv
    sr_ref[0] = msum(re_ref[0]) * inv
    iv_ref[0] = inv


def _special(xfr, xrr, masks, cnts):
    return pl.pallas_call(
        _special_body,
        grid=(_B,),
        in_specs=[pl.BlockSpec((1, _NS, _D), lambda b: (b, 0, 0)),
                  pl.BlockSpec((1, _NS, _L), lambda b: (b, 0, 0)),
                  pl.BlockSpec((1, _NS, 4), lambda b: (b, 0, 0)),
                  pl.BlockSpec((1, 4, 1), lambda b: (b, 0, 0))],
        out_specs=[pl.BlockSpec((1, 4, _D), lambda b: (b, 0, 0)),
                   pl.BlockSpec((1, 4, _L), lambda b: (b, 0, 0)),
                   pl.BlockSpec((1, 4, 1), lambda b: (b, 0, 0))],
        out_shape=[jax.ShapeDtypeStruct((_B, 4, _D), jnp.float32),
                   jax.ShapeDtypeStruct((_B, 4, _L), jnp.float32),
                   jax.ShapeDtypeStruct((_B, 4, 1), jnp.float32)],
    )(xfr, xrr, masks, cnts)


def _sc_gather(xf2d, xr2d, idx2d):
    mesh = plsc.VectorSubcoreMesh(core_axis_name="c", subcore_axis_name="s")

    @functools.partial(
        pl.kernel,
        mesh=mesh,
        out_type=(jax.ShapeDtypeStruct((_B, _NNEW, _D), jnp.float32),
                  jax.ShapeDtypeStruct((_B, _NNEW, _L), jnp.float32)),
        scratch_types=[pltpu.VMEM((32,), jnp.int32),
                       pltpu.VMEM((32, _D), jnp.float32),
                       pltpu.VMEM((32, _L), jnp.float32),
                       pltpu.SemaphoreType.DMA,
                       pltpu.SemaphoreType.DMA],
    )
    def gk(xf_hbm, xr_hbm, idx_hbm, of_hbm, or_hbm,
           idx_v, rf_v, rr_v, smf, smr):
        wid = lax.axis_index("s") * 2 + lax.axis_index("c")
        b = wid // 2
        half = wid - 2 * b

        @pl.loop(0, 23)
        def _(j):
            c = half + 2 * j

            @pl.when(c < 45)
            def _():
                base = c * 32
                pltpu.sync_copy(idx_hbm.at[pl.ds(b * 1440 + base, 32)],
                                idx_v)
                h1 = pltpu.async_copy(xf_hbm.at[idx_v], rf_v, smf)
                h2 = pltpu.async_copy(xr_hbm.at[idx_v], rr_v, smr)
                h1.wait()
                h2.wait()

                @pl.when(c < 44)
                def _():
                    pltpu.sync_copy(rf_v, of_hbm.at[b].at[pl.ds(base, 32)])
                    pltpu.sync_copy(rr_v, or_hbm.at[b].at[pl.ds(base, 32)])

                @pl.when(c == 44)
                def _():
                    @pl.loop(0, 26)
                    def _(r):
                        pltpu.sync_copy(rf_v.at[r], of_hbm.at[b, 1408 + r])
                        pltpu.sync_copy(rr_v.at[r], or_hbm.at[b, 1408 + r])

    return gk(xf2d, xr2d, idx2d.reshape(_B * 1440))


def _overwrite_body(sl_ref, dg_ref, gf_ref, gr_ref, xf_ref, xr_ref,
                    sdf_ref, sdr_ref, iv_ref, of_ref, or_ref,
                    xfs, xrs, ofs, ors, sma, smb, smc, smd):
    del gf_ref, gr_ref
    n = _B * 4

    def fire_in(g, _):
        d = dg_ref[g]
        pltpu.make_async_copy(xf_ref.at[d], xfs.at[g], sma).start()
        pltpu.make_async_copy(xr_ref.at[d], xrs.at[g], smb).start()
        return 0

    jax.lax.fori_loop(0, n, fire_in, 0)

    def drain_in(g, _):
        pltpu.make_async_copy(xf_ref.at[0], xfs.at[0], sma).wait()
        pltpu.make_async_copy(xr_ref.at[0], xrs.at[0], smb).wait()
        return 0

    jax.lax.fori_loop(0, n, drain_in, 0)

    ofs[...] = xfs[...] * iv_ref[...] + sdf_ref[...]
    ors[...] = xrs[...] * iv_ref[...] + sdr_ref[...]

    def fire_out(g, _):
        b = g // 4
        s = sl_ref[g]
        pltpu.make_async_copy(ofs.at[g], of_ref.at[b, s], smc).start()
        pltpu.make_async_copy(ors.at[g], or_ref.at[b, s], smd).start()
        return 0

    jax.lax.fori_loop(0, n, fire_out, 0)

    def drain_out(g, _):
        pltpu.make_async_copy(ofs.at[0], of_ref.at[0, 0], smc).wait()
        pltpu.make_async_copy(ors.at[0], or_ref.at[0, 0], smd).wait()
        return 0

    jax.lax.fori_loop(0, n, drain_out, 0)


def _overwrite(slots_flat, dstg_flat, gf, gr, xf2d, xr2d, sdf, sdr, inv):
    anyspace = pl.BlockSpec(memory_space=pltpu.MemorySpace.HBM)
    grid_spec = pltpu.PrefetchScalarGridSpec(
        num_scalar_prefetch=2,
        grid=(1,),
        in_specs=[anyspace, anyspace, anyspace, anyspace,
                  pl.BlockSpec((_B * 4, _D), lambda g, sl, dg: (0, 0)),
                  pl.BlockSpec((_B * 4, _L), lambda g, sl, dg: (0, 0)),
                  pl.BlockSpec((_B * 4, 1), lambda g, sl, dg: (0, 0))],
        out_specs=[anyspace, anyspace],
        scratch_shapes=[pltpu.VMEM((_B * 4, _D), jnp.float32),
                        pltpu.VMEM((_B * 4, _L), jnp.float32),
                        pltpu.VMEM((_B * 4, _D), jnp.float32),
                        pltpu.VMEM((_B * 4, _L), jnp.float32),
                        pltpu.SemaphoreType.DMA, pltpu.SemaphoreType.DMA,
                        pltpu.SemaphoreType.DMA, pltpu.SemaphoreType.DMA],
    )
    return pl.pallas_call(
        _overwrite_body,
        grid_spec=grid_spec,
        out_shape=[jax.ShapeDtypeStruct((_B, _NNEW, _D), jnp.float32),
                   jax.ShapeDtypeStruct((_B, _NNEW, _L), jnp.float32)],
        input_output_aliases={2: 0, 3: 1},
    )(slots_flat, dstg_flat, gf, gr, xf2d, xr2d, sdf, sdr, inv)


@jax.jit
def kernel(x_feat, x_raw):
    xfr = x_feat.reshape(_B, _NS, 2 * _D)
    xrr = x_raw.reshape(_B, _NS, 2 * _L)

    me, mo = _metric(xfr)
    ue, uo, pglob, masks, cnts, slots, dstg = _plan(me, mo)
    unpool = jnp.concatenate([ue, uo], axis=2).reshape(_B, _N)

    sdf, sdr, inv = _special(xfr, xrr, masks, cnts)

    idx2d = jnp.pad(pglob.reshape(_B, _NNEW), ((0, 0), (0, 1440 - _NNEW)))
    gf, gr = _sc_gather(x_feat.reshape(_B * _N, _D),
                        x_raw.reshape(_B * _N, _L),
                        idx2d)

    of, orr = _overwrite(slots.reshape(_B * 4), dstg.reshape(_B * 4),
                         gf, gr,
                         x_feat.reshape(_B * _N, _D),
                         x_raw.reshape(_B * _N, _L),
                         sdf.reshape(_B * 4, _D), sdr.reshape(_B * 4, _L),
                         inv.reshape(_B * 4, 1))

    return (of, orr, unpool)
